# initial kernel scaffold (unmeasured)
import jax
import jax.numpy as jnp
from jax import lax
from jax.experimental import pallas as pl
from jax.experimental.pallas import tpu as pltpu


def kernel(
    x,
):
    def body(*refs):
        pass

    out_shape = jax.ShapeDtypeStruct(..., jnp.float32)
    return pl.pallas_call(body, out_shape=out_shape)(...)


# baseline (device time: 2129625 ns/iter reference)
import jax
import jax.numpy as jnp
from jax import lax
from jax.experimental import pallas as pl
from jax.experimental.pallas import tpu as pltpu


def kernel(x):
    m, n = x.shape
    n_half = n // 2

    def body(x_ref, out_ref, local_sem, send_sem, recv_sem):
        my_x = lax.axis_index("x")
        my_y = lax.axis_index("y")
        my_z = lax.axis_index("z")
        partner = (my_x, my_y, 1 - my_z)

        barrier_sem = pltpu.get_barrier_semaphore()
        pl.semaphore_signal(
            barrier_sem, inc=1,
            device_id=partner, device_id_type=pl.DeviceIdType.MESH,
        )
        pl.semaphore_wait(barrier_sem, 1)

        local = pltpu.make_async_copy(
            x_ref.at[:, pl.ds(my_z * n_half, n_half)],
            out_ref.at[pl.ds(my_z * m, m), :],
            local_sem,
        )
        local.start()

        rdma = pltpu.make_async_remote_copy(
            src_ref=x_ref.at[:, pl.ds((1 - my_z) * n_half, n_half)],
            dst_ref=out_ref.at[pl.ds(my_z * m, m), :],
            send_sem=send_sem,
            recv_sem=recv_sem,
            device_id=partner,
            device_id_type=pl.DeviceIdType.MESH,
        )
        rdma.start()

        local.wait()
        rdma.wait()

    return pl.pallas_call(
        body,
        out_shape=jax.ShapeDtypeStruct((2 * m, n_half), x.dtype),
        in_specs=[pl.BlockSpec(memory_space=pl.ANY)],
        out_specs=pl.BlockSpec(memory_space=pl.ANY),
        scratch_shapes=[
            pltpu.SemaphoreType.DMA,
            pltpu.SemaphoreType.DMA,
            pltpu.SemaphoreType.DMA,
        ],
        compiler_params=pltpu.CompilerParams(collective_id=0),
    )(x)


# device time: 813343 ns/iter; 2.6184x vs baseline; 2.6184x over previous
import jax
import jax.numpy as jnp
from jax import lax
from jax.experimental import pallas as pl
from jax.experimental.pallas import tpu as pltpu

N_CHUNK = 8


def kernel(x):
    m, n = x.shape
    n_half = n // 2
    c = m // N_CHUNK

    def body(x_ref, out_ref, lbuf, sbuf, lsem, lout_sems, ssem,
             send_sems, recv_sems):
        my_x = lax.axis_index("x")
        my_y = lax.axis_index("y")
        my_z = lax.axis_index("z")
        partner = (my_x, my_y, 1 - my_z)

        barrier_sem = pltpu.get_barrier_semaphore()
        pl.semaphore_signal(
            barrier_sem, inc=1,
            device_id=partner, device_id_type=pl.DeviceIdType.MESH,
        )
        pl.semaphore_wait(barrier_sem, 1)

        send_descs = []
        local_outs = {}
        for i in range(N_CHUNK):
            b = i % 2

            if i >= 2:
                send_descs[i - 2].wait_send()
            cin = pltpu.make_async_copy(
                x_ref.at[pl.ds(i * c, c), pl.ds((1 - my_z) * n_half, n_half)],
                sbuf.at[b],
                ssem,
            )
            cin.start()
            cin.wait()
            sd = pltpu.make_async_remote_copy(
                src_ref=sbuf.at[b],
                dst_ref=out_ref.at[pl.ds(my_z * m + i * c, c), :],
                send_sem=send_sems.at[i],
                recv_sem=recv_sems.at[i],
                device_id=partner,
                device_id_type=pl.DeviceIdType.MESH,
            )
            sd.start()
            send_descs.append(sd)

            if b in local_outs:
                local_outs[b].wait()
            lin = pltpu.make_async_copy(
                x_ref.at[pl.ds(i * c, c), pl.ds(my_z * n_half, n_half)],
                lbuf.at[b],
                lsem,
            )
            lin.start()
            lin.wait()
            lout = pltpu.make_async_copy(
                lbuf.at[b],
                out_ref.at[pl.ds(my_z * m + i * c, c), :],
                lout_sems.at[b],
            )
            lout.start()
            local_outs[b] = lout

        for sd in send_descs[-2:]:
            sd.wait_send()
        for lout in local_outs.values():
            lout.wait()

        for i in range(N_CHUNK):
            rd = pltpu.make_async_remote_copy(
                src_ref=sbuf.at[0],
                dst_ref=out_ref.at[pl.ds((1 - my_z) * m + i * c, c), :],
                send_sem=send_sems.at[i],
                recv_sem=recv_sems.at[i],
                device_id=partner,
                device_id_type=pl.DeviceIdType.MESH,
            )
            rd.wait_recv()

    return pl.pallas_call(
        body,
        out_shape=jax.ShapeDtypeStruct((2 * m, n_half), x.dtype),
        in_specs=[pl.BlockSpec(memory_space=pl.ANY)],
        out_specs=pl.BlockSpec(memory_space=pl.ANY),
        scratch_shapes=[
            pltpu.VMEM((2, c, n_half), jnp.float32),
            pltpu.VMEM((2, c, n_half), jnp.float32),
            pltpu.SemaphoreType.DMA,
            pltpu.SemaphoreType.DMA((2,)),
            pltpu.SemaphoreType.DMA,
            pltpu.SemaphoreType.DMA((N_CHUNK,)),
            pltpu.SemaphoreType.DMA((N_CHUNK,)),
        ],
        compiler_params=pltpu.CompilerParams(collective_id=0),
    )(x)
